# baseline (speedup 1.0000x reference)
import jax
import jax.numpy as jnp
from jax import lax
from jax.experimental import pallas as pl
from jax.experimental.pallas import tpu as pltpu
from jax.experimental.pallas import tpu_sc as plsc

N = 10000
NP = 10240
E = 320000
EP = 327680
D = 128
OUT = 2
NC = 2
NS = 16
W = 128
ROWS_T = EP // (NS * W)
SEGR = 8
NSEG = ROWS_T // SEGR
RPT = NP // NS
DUMP = RPT // W
DUMMY = NP - 1
NBUF = 2

_mesh = plsc.VectorSubcoreMesh(core_axis_name="c", subcore_axis_name="s",
                               num_cores=NC, num_subcores=NS)


def _acc_body(table, esrc, edst, et, acc_out, acc_s,
              seg_src, seg_dst, seg_typ,
              buf0, buf1, sem0, sem1):
    r = lax.axis_index("c")
    s = lax.axis_index("s")
    row0 = s * RPT
    bufs = (buf0, buf1)
    sems = (sem0, sem1)
    rows_v = buf0

    def zrow(i, c):
        for k in range(D // 16):
            rows_v[i, pl.ds(k * 16, 16)] = jnp.zeros((16,), jnp.float32)
        return c
    lax.fori_loop(0, W, zrow, 0)
    for j in range(DUMP):
        pltpu.sync_copy(rows_v, acc_s.at[pl.ds(row0 + j * W, W)])
    plsc.subcore_barrier()

    tbase = s * ROWS_T

    def seg_body(g, c):
        rb = tbase + g * SEGR
        pltpu.sync_copy(esrc.at[pl.ds(rb, SEGR)], seg_src)
        pltpu.sync_copy(edst.at[pl.ds(rb, SEGR)], seg_dst)
        pltpu.sync_copy(et.at[pl.ds(rb, SEGR)], seg_typ)

        def tbody(i, c2):
            for k in range(W // 16):
                sl = pl.ds(k * 16, 16)
                msk = seg_typ[i, sl] == r
                seg_src[i, sl] = jnp.where(msk, seg_src[i, sl] + r * NP, 0)
                seg_dst[i, sl] = jnp.where(msk, seg_dst[i, sl], DUMMY)
            return c2
        lax.fori_loop(0, SEGR, tbody, 0)

        handles = [
            pltpu.async_copy(table.at[seg_src.at[j]], bufs[j], sems[j])
            for j in range(NBUF)
        ]
        for j in range(SEGR):
            b = j % NBUF
            handles[b].wait()
            pltpu.sync_copy(bufs[b], acc_s.at[seg_dst.at[j]], add=True)
            nj = j + NBUF
            if nj < SEGR:
                handles[b] = pltpu.async_copy(
                    table.at[seg_src.at[nj]], bufs[b], sems[b])
        return c
    lax.fori_loop(0, NSEG, seg_body, 0)
    plsc.subcore_barrier()

    obase = r * NP + row0
    for j in range(DUMP):
        pltpu.sync_copy(acc_s.at[pl.ds(row0 + j * W, W)], rows_v)
        pltpu.sync_copy(rows_v, acc_out.at[pl.ds(obase + j * W, W)])


def _cnt_body(edst, et, cnt_out, cnt_s, seg_dst, seg_typ, ones_v, gsem):
    r = lax.axis_index("c")
    s = lax.axis_index("s")
    row0 = s * RPT

    def zrow(i, c):
        for k in range(D // 16):
            ones_v[i, pl.ds(k * 16, 16)] = jnp.zeros((16,), jnp.float32)
        return c
    lax.fori_loop(0, W, zrow, 0)
    for j in range(DUMP):
        pltpu.sync_copy(ones_v, cnt_s.at[pl.ds(row0 + j * W, W)])

    def orow(i, c):
        for k in range(D // 16):
            ones_v[i, pl.ds(k * 16, 16)] = jnp.ones((16,), jnp.float32)
        return c
    lax.fori_loop(0, W, orow, 0)
    plsc.subcore_barrier()

    tbase = s * ROWS_T

    def seg_body(g, c):
        rb = tbase + g * SEGR
        pltpu.sync_copy(edst.at[pl.ds(rb, SEGR)], seg_dst)
        pltpu.sync_copy(et.at[pl.ds(rb, SEGR)], seg_typ)

        def tbody(i, c2):
            for k in range(W // 16):
                sl = pl.ds(k * 16, 16)
                msk = seg_typ[i, sl] == r
                seg_dst[i, sl] = jnp.where(msk, seg_dst[i, sl], DUMMY)
            return c2
        lax.fori_loop(0, SEGR, tbody, 0)

        def mbody(j, c2):
            pltpu.sync_copy(ones_v, cnt_s.at[seg_dst.at[j]], add=True)
            return c2
        lax.fori_loop(0, SEGR, mbody, 0)
        return c
    lax.fori_loop(0, NSEG, seg_body, 0)
    plsc.subcore_barrier()

    obase = r * NP + row0
    for j in range(DUMP):
        pltpu.sync_copy(cnt_s.at[pl.ds(row0 + j * W, W)], ones_v)
        pltpu.sync_copy(ones_v, cnt_out.at[pl.ds(obase + j * W, W)])


_acc_pass = pl.kernel(
    _acc_body,
    out_type=jax.ShapeDtypeStruct((NC * NP, D), jnp.float32),
    mesh=_mesh,
    scratch_types=(
        [pltpu.VMEM_SHARED((NP, D), jnp.float32)]
        + [pltpu.VMEM((SEGR, W), jnp.int32)] * 3
        + [pltpu.VMEM((W, D), jnp.float32)] * NBUF
        + [pltpu.SemaphoreType.DMA] * NBUF
    ))

_cnt_pass = pl.kernel(
    _cnt_body,
    out_type=jax.ShapeDtypeStruct((NC * NP, D), jnp.float32),
    mesh=_mesh,
    scratch_types=[
        pltpu.VMEM_SHARED((NP, D), jnp.float32),
        pltpu.VMEM((SEGR, W), jnp.int32),
        pltpu.VMEM((SEGR, W), jnp.int32),
        pltpu.VMEM((W, D), jnp.float32),
        pltpu.SemaphoreType.DMA,
    ])


BT = 512
GRID = NP // BT
_f32 = jnp.float32


def _mm1_body(x_ref, r1_ref, w1_ref, b1_ref, base_ref, tab_ref):
    xb = x_ref[...]
    base_ref[...] = (jnp.dot(xb, r1_ref[...], preferred_element_type=_f32)
                     + b1_ref[...])
    tab_ref[0] = jnp.dot(xb, w1_ref[0], preferred_element_type=_f32)
    tab_ref[1] = jnp.dot(xb, w1_ref[1], preferred_element_type=_f32)


def _mid_body(base_ref, acc_ref, cnt_ref, r2_ref, w2_ref, b2_ref,
              base2_ref, tab2_ref):
    inv0 = 1.0 / jnp.maximum(cnt_ref[0], 1.0)
    inv1 = 1.0 / jnp.maximum(cnt_ref[1], 1.0)
    h = jax.nn.relu(base_ref[...] + inv0 * acc_ref[0] + inv1 * acc_ref[1])
    base2_ref[...] = (jnp.dot(h, r2_ref[...], preferred_element_type=_f32)
                      + b2_ref[...])
    tab2_ref[0] = jnp.dot(h, w2_ref[0], preferred_element_type=_f32)
    tab2_ref[1] = jnp.dot(h, w2_ref[1], preferred_element_type=_f32)


def _fin_body(base_ref, acc_ref, cnt_ref, wc_ref, bc_ref, out_ref):
    inv0 = 1.0 / jnp.maximum(cnt_ref[0], 1.0)
    inv1 = 1.0 / jnp.maximum(cnt_ref[1], 1.0)
    h = jax.nn.relu(base_ref[...] + inv0 * acc_ref[0] + inv1 * acc_ref[1])
    out_ref[...] = (jnp.dot(h, wc_ref[...], preferred_element_type=_f32)
                    + bc_ref[...])


def _mm1_call(xp, R1, W1, b1p):
    return pl.pallas_call(
        _mm1_body,
        grid=(GRID,),
        in_specs=[
            pl.BlockSpec((BT, D), lambda i: (i, 0)),
            pl.BlockSpec((D, D), lambda i: (0, 0)),
            pl.BlockSpec((NC, D, D), lambda i: (0, 0, 0)),
            pl.BlockSpec((1, D), lambda i: (0, 0)),
        ],
        out_specs=[
            pl.BlockSpec((BT, D), lambda i: (i, 0)),
            pl.BlockSpec((NC, BT, D), lambda i: (0, i, 0)),
        ],
        out_shape=[
            jax.ShapeDtypeStruct((NP, D), _f32),
            jax.ShapeDtypeStruct((NC, NP, D), _f32),
        ],
    )(xp, R1, W1, b1p)


def _mid_call(base1, acc1, cnt, R2, W2, b2p):
    return pl.pallas_call(
        _mid_body,
        grid=(GRID,),
        in_specs=[
            pl.BlockSpec((BT, D), lambda i: (i, 0)),
            pl.BlockSpec((NC, BT, D), lambda i: (0, i, 0)),
            pl.BlockSpec((NC, BT, D), lambda i: (0, i, 0)),
            pl.BlockSpec((D, D), lambda i: (0, 0)),
            pl.BlockSpec((NC, D, D), lambda i: (0, 0, 0)),
            pl.BlockSpec((1, D), lambda i: (0, 0)),
        ],
        out_specs=[
            pl.BlockSpec((BT, D), lambda i: (i, 0)),
            pl.BlockSpec((NC, BT, D), lambda i: (0, i, 0)),
        ],
        out_shape=[
            jax.ShapeDtypeStruct((NP, D), _f32),
            jax.ShapeDtypeStruct((NC, NP, D), _f32),
        ],
    )(base1, acc1, cnt, R2, W2, b2p)


def _fin_call(base2, acc2, cnt, Wcp, bcp):
    return pl.pallas_call(
        _fin_body,
        grid=(GRID,),
        in_specs=[
            pl.BlockSpec((BT, D), lambda i: (i, 0)),
            pl.BlockSpec((NC, BT, D), lambda i: (0, i, 0)),
            pl.BlockSpec((NC, BT, D), lambda i: (0, i, 0)),
            pl.BlockSpec((D, D), lambda i: (0, 0)),
            pl.BlockSpec((1, D), lambda i: (0, 0)),
        ],
        out_specs=pl.BlockSpec((BT, D), lambda i: (i, 0)),
        out_shape=jax.ShapeDtypeStruct((NP, D), _f32),
    )(base2, acc2, cnt, Wcp, bcp)


def kernel(x, edge_index, edge_type, W1, R1, b1, W2, R2, b2, Wc, bc):
    xp = jnp.pad(x, ((0, NP - N), (0, 0)))
    b1p = b1.reshape(1, D)
    b2p = b2.reshape(1, D)
    Wcp = jnp.pad(Wc, ((0, 0), (0, D - OUT)))
    bcp = jnp.pad(bc, (0, D - OUT)).reshape(1, D)
    esrc = jnp.pad(edge_index[0].astype(jnp.int32),
                   (0, EP - E)).reshape(EP // W, W)
    edst = jnp.pad(edge_index[1].astype(jnp.int32),
                   (0, EP - E)).reshape(EP // W, W)
    et = jnp.pad(edge_type.astype(jnp.int32), (0, EP - E),
                 constant_values=NC).reshape(EP // W, W)

    cnt = _cnt_pass(edst, et).reshape(NC, NP, D)
    base1, tab1 = _mm1_call(xp, R1, W1, b1p)
    acc1 = _acc_pass(tab1.reshape(NC * NP, D), esrc, edst, et)
    base2, tab2 = _mid_call(base1, acc1.reshape(NC, NP, D),
                            cnt, R2, W2, b2p)
    acc2 = _acc_pass(tab2.reshape(NC * NP, D), esrc, edst, et)
    outp = _fin_call(base2, acc2.reshape(NC, NP, D), cnt, Wcp, bcp)
    return outp[:N, :OUT]


# speedup vs baseline: 1.0026x; 1.0026x over previous
import jax
import jax.numpy as jnp
from jax import lax
from jax.experimental import pallas as pl
from jax.experimental.pallas import tpu as pltpu
from jax.experimental.pallas import tpu_sc as plsc

N = 10000
NP = 10240
E = 320000
EP = 327680
D = 128
OUT = 2
NC = 2
NS = 16
W = 128
ROWS_T = EP // (NS * W)
SEGR = 8
NSEG = ROWS_T // SEGR
RPT = NP // NS
DUMP = RPT // W
NBUF = 2

_mesh = plsc.VectorSubcoreMesh(core_axis_name="c", subcore_axis_name="s",
                               num_cores=NC, num_subcores=NS)


def _acc_body(table, esrc, edst, et, edmy, acc_out, acc_s,
              seg_src, seg_dst, seg_typ, seg_dmy,
              buf0, buf1, sem0, sem1):
    r = lax.axis_index("c")
    s = lax.axis_index("s")
    row0 = s * RPT
    bufs = (buf0, buf1)
    sems = (sem0, sem1)
    rows_v = buf0
    pltpu.sync_copy(edmy, seg_dmy)

    def zrow(i, c):
        for k in range(D // 16):
            rows_v[i, pl.ds(k * 16, 16)] = jnp.zeros((16,), jnp.float32)
        return c
    lax.fori_loop(0, W, zrow, 0)
    for j in range(DUMP):
        pltpu.sync_copy(rows_v, acc_s.at[pl.ds(row0 + j * W, W)])
    plsc.subcore_barrier()

    tbase = s * ROWS_T

    def seg_body(g, c):
        rb = tbase + g * SEGR
        pltpu.sync_copy(esrc.at[pl.ds(rb, SEGR)], seg_src)
        pltpu.sync_copy(edst.at[pl.ds(rb, SEGR)], seg_dst)
        pltpu.sync_copy(et.at[pl.ds(rb, SEGR)], seg_typ)

        def tbody(i, c2):
            for k in range(W // 16):
                sl = pl.ds(k * 16, 16)
                msk = seg_typ[i, sl] == r
                seg_src[i, sl] = jnp.where(msk, seg_src[i, sl] + r * NP, 0)
                seg_dst[i, sl] = jnp.where(msk, seg_dst[i, sl],
                                           seg_dmy[i, sl])
            return c2
        lax.fori_loop(0, SEGR, tbody, 0)

        handles = [
            pltpu.async_copy(table.at[seg_src.at[j]], bufs[j], sems[j])
            for j in range(NBUF)
        ]
        for j in range(SEGR):
            b = j % NBUF
            handles[b].wait()
            pltpu.sync_copy(bufs[b], acc_s.at[seg_dst.at[j]], add=True)
            nj = j + NBUF
            if nj < SEGR:
                handles[b] = pltpu.async_copy(
                    table.at[seg_src.at[nj]], bufs[b], sems[b])
        return c
    lax.fori_loop(0, NSEG, seg_body, 0)
    plsc.subcore_barrier()

    obase = r * NP + row0
    for j in range(DUMP):
        pltpu.sync_copy(acc_s.at[pl.ds(row0 + j * W, W)], rows_v)
        pltpu.sync_copy(rows_v, acc_out.at[pl.ds(obase + j * W, W)])


def _cnt_body(edst, et, edmy, cnt_out, cnt_s, seg_dst, seg_typ, seg_dmy,
              ones_v, gsem):
    r = lax.axis_index("c")
    s = lax.axis_index("s")
    row0 = s * RPT
    pltpu.sync_copy(edmy, seg_dmy)

    def zrow(i, c):
        for k in range(D // 16):
            ones_v[i, pl.ds(k * 16, 16)] = jnp.zeros((16,), jnp.float32)
        return c
    lax.fori_loop(0, W, zrow, 0)
    for j in range(DUMP):
        pltpu.sync_copy(ones_v, cnt_s.at[pl.ds(row0 + j * W, W)])

    def orow(i, c):
        for k in range(D // 16):
            ones_v[i, pl.ds(k * 16, 16)] = jnp.ones((16,), jnp.float32)
        return c
    lax.fori_loop(0, W, orow, 0)
    plsc.subcore_barrier()

    tbase = s * ROWS_T

    def seg_body(g, c):
        rb = tbase + g * SEGR
        pltpu.sync_copy(edst.at[pl.ds(rb, SEGR)], seg_dst)
        pltpu.sync_copy(et.at[pl.ds(rb, SEGR)], seg_typ)

        def tbody(i, c2):
            for k in range(W // 16):
                sl = pl.ds(k * 16, 16)
                msk = seg_typ[i, sl] == r
                seg_dst[i, sl] = jnp.where(msk, seg_dst[i, sl],
                                           seg_dmy[i, sl])
            return c2
        lax.fori_loop(0, SEGR, tbody, 0)

        def mbody(j, c2):
            pltpu.sync_copy(ones_v, cnt_s.at[seg_dst.at[j]], add=True)
            return c2
        lax.fori_loop(0, SEGR, mbody, 0)
        return c
    lax.fori_loop(0, NSEG, seg_body, 0)
    plsc.subcore_barrier()

    obase = r * NP + row0
    for j in range(DUMP):
        pltpu.sync_copy(cnt_s.at[pl.ds(row0 + j * W, W)], ones_v)
        pltpu.sync_copy(ones_v, cnt_out.at[pl.ds(obase + j * W, W)])


_acc_pass = pl.kernel(
    _acc_body,
    out_type=jax.ShapeDtypeStruct((NC * NP, D), jnp.float32),
    mesh=_mesh,
    scratch_types=(
        [pltpu.VMEM_SHARED((NP, D), jnp.float32)]
        + [pltpu.VMEM((SEGR, W), jnp.int32)] * 4
        + [pltpu.VMEM((W, D), jnp.float32)] * NBUF
        + [pltpu.SemaphoreType.DMA] * NBUF
    ))

_cnt_pass = pl.kernel(
    _cnt_body,
    out_type=jax.ShapeDtypeStruct((NC * NP, D), jnp.float32),
    mesh=_mesh,
    scratch_types=[
        pltpu.VMEM_SHARED((NP, D), jnp.float32),
        pltpu.VMEM((SEGR, W), jnp.int32),
        pltpu.VMEM((SEGR, W), jnp.int32),
        pltpu.VMEM((SEGR, W), jnp.int32),
        pltpu.VMEM((W, D), jnp.float32),
        pltpu.SemaphoreType.DMA,
    ])


BT = 512
GRID = NP // BT
_f32 = jnp.float32


def _mm1_body(x_ref, r1_ref, w1_ref, b1_ref, base_ref, tab_ref):
    xb = x_ref[...]
    base_ref[...] = (jnp.dot(xb, r1_ref[...], preferred_element_type=_f32)
                     + b1_ref[...])
    tab_ref[0] = jnp.dot(xb, w1_ref[0], preferred_element_type=_f32)
    tab_ref[1] = jnp.dot(xb, w1_ref[1], preferred_element_type=_f32)


def _mid_body(base_ref, acc_ref, cnt_ref, r2_ref, w2_ref, b2_ref,
              base2_ref, tab2_ref):
    inv0 = 1.0 / jnp.maximum(cnt_ref[0], 1.0)
    inv1 = 1.0 / jnp.maximum(cnt_ref[1], 1.0)
    h = jax.nn.relu(base_ref[...] + inv0 * acc_ref[0] + inv1 * acc_ref[1])
    base2_ref[...] = (jnp.dot(h, r2_ref[...], preferred_element_type=_f32)
                      + b2_ref[...])
    tab2_ref[0] = jnp.dot(h, w2_ref[0], preferred_element_type=_f32)
    tab2_ref[1] = jnp.dot(h, w2_ref[1], preferred_element_type=_f32)


def _fin_body(base_ref, acc_ref, cnt_ref, wc_ref, bc_ref, out_ref):
    inv0 = 1.0 / jnp.maximum(cnt_ref[0], 1.0)
    inv1 = 1.0 / jnp.maximum(cnt_ref[1], 1.0)
    h = jax.nn.relu(base_ref[...] + inv0 * acc_ref[0] + inv1 * acc_ref[1])
    out_ref[...] = (jnp.dot(h, wc_ref[...], preferred_element_type=_f32)
                    + bc_ref[...])


def _mm1_call(xp, R1, W1, b1p):
    return pl.pallas_call(
        _mm1_body,
        grid=(GRID,),
        in_specs=[
            pl.BlockSpec((BT, D), lambda i: (i, 0)),
            pl.BlockSpec((D, D), lambda i: (0, 0)),
            pl.BlockSpec((NC, D, D), lambda i: (0, 0, 0)),
            pl.BlockSpec((1, D), lambda i: (0, 0)),
        ],
        out_specs=[
            pl.BlockSpec((BT, D), lambda i: (i, 0)),
            pl.BlockSpec((NC, BT, D), lambda i: (0, i, 0)),
        ],
        out_shape=[
            jax.ShapeDtypeStruct((NP, D), _f32),
            jax.ShapeDtypeStruct((NC, NP, D), _f32),
        ],
    )(xp, R1, W1, b1p)


def _mid_call(base1, acc1, cnt, R2, W2, b2p):
    return pl.pallas_call(
        _mid_body,
        grid=(GRID,),
        in_specs=[
            pl.BlockSpec((BT, D), lambda i: (i, 0)),
            pl.BlockSpec((NC, BT, D), lambda i: (0, i, 0)),
            pl.BlockSpec((NC, BT, D), lambda i: (0, i, 0)),
            pl.BlockSpec((D, D), lambda i: (0, 0)),
            pl.BlockSpec((NC, D, D), lambda i: (0, 0, 0)),
            pl.BlockSpec((1, D), lambda i: (0, 0)),
        ],
        out_specs=[
            pl.BlockSpec((BT, D), lambda i: (i, 0)),
            pl.BlockSpec((NC, BT, D), lambda i: (0, i, 0)),
        ],
        out_shape=[
            jax.ShapeDtypeStruct((NP, D), _f32),
            jax.ShapeDtypeStruct((NC, NP, D), _f32),
        ],
    )(base1, acc1, cnt, R2, W2, b2p)


def _fin_call(base2, acc2, cnt, Wcp, bcp):
    return pl.pallas_call(
        _fin_body,
        grid=(GRID,),
        in_specs=[
            pl.BlockSpec((BT, D), lambda i: (i, 0)),
            pl.BlockSpec((NC, BT, D), lambda i: (0, i, 0)),
            pl.BlockSpec((NC, BT, D), lambda i: (0, i, 0)),
            pl.BlockSpec((D, D), lambda i: (0, 0)),
            pl.BlockSpec((1, D), lambda i: (0, 0)),
        ],
        out_specs=pl.BlockSpec((BT, D), lambda i: (i, 0)),
        out_shape=jax.ShapeDtypeStruct((NP, D), _f32),
    )(base2, acc2, cnt, Wcp, bcp)


def kernel(x, edge_index, edge_type, W1, R1, b1, W2, R2, b2, Wc, bc):
    xp = jnp.pad(x, ((0, NP - N), (0, 0)))
    b1p = b1.reshape(1, D)
    b2p = b2.reshape(1, D)
    Wcp = jnp.pad(Wc, ((0, 0), (0, D - OUT)))
    bcp = jnp.pad(bc, (0, D - OUT)).reshape(1, D)
    esrc = jnp.pad(edge_index[0].astype(jnp.int32),
                   (0, EP - E)).reshape(EP // W, W)
    edst = jnp.pad(edge_index[1].astype(jnp.int32),
                   (0, EP - E)).reshape(EP // W, W)
    et = jnp.pad(edge_type.astype(jnp.int32), (0, EP - E),
                 constant_values=NC).reshape(EP // W, W)
    edmy = (N + (jnp.arange(SEGR * W, dtype=jnp.int32) % (NP - N))
            ).reshape(SEGR, W)

    cnt = _cnt_pass(edst, et, edmy).reshape(NC, NP, D)
    base1, tab1 = _mm1_call(xp, R1, W1, b1p)
    acc1 = _acc_pass(tab1.reshape(NC * NP, D), esrc, edst, et, edmy)
    base2, tab2 = _mid_call(base1, acc1.reshape(NC, NP, D),
                            cnt, R2, W2, b2p)
    acc2 = _acc_pass(tab2.reshape(NC * NP, D), esrc, edst, et, edmy)
    outp = _fin_call(base2, acc2.reshape(NC, NP, D), cnt, Wcp, bcp)
    return outp[:N, :OUT]
